# baseline (speedup 1.0000x reference)
import math

import jax
import jax.numpy as jnp
from jax import lax
from jax.experimental import pallas as pl
from jax.experimental.pallas import tpu as pltpu

_NB = 32
_NOUT = 3 * _NB + 1
_BOUND = 10.0
_MIN_W = 1e-05
_MIN_H = 1e-05
_MIN_D = 1e-05
_L2PI = 0.5 * math.log(2.0 * math.pi)
_PART = 16
_ROWS = 4096


def _flow_block(c_ref, w_ref, b_ref, x_ref, s_ref, o_ref):
    ne, npart, dims_c = c_ref.shape
    rows = ne * npart
    cb = c_ref[...].reshape(rows, dims_c)
    theta = lax.dot_general(
        w_ref[...], cb,
        dimension_numbers=(((0,), (1,)), ((), ())),
        preferred_element_type=jnp.float32,
    ) + b_ref[...]

    uw = theta[0:_NB, :]
    uh = theta[_NB:2 * _NB, :]
    ud = theta[2 * _NB:_NOUT, :]

    xrow = x_ref[0]
    inside = (xrow >= -_BOUND) & (xrow <= _BOUND)
    xq = jnp.clip(xrow, -_BOUND, _BOUND)

    def edges(u, min_v):
        e = jnp.exp(u)
        scale = (1.0 - min_v * _NB) / jnp.sum(e, axis=0, keepdims=True)
        s = min_v + e * scale
        cum = s
        for k in (1, 2, 4, 8, 16):
            cum = cum + jnp.concatenate(
                [jnp.zeros((k, rows), jnp.float32), cum[:_NB - k, :]], axis=0)
        cum = 2.0 * _BOUND * cum - _BOUND
        rid = lax.broadcasted_iota(jnp.int32, (_NB, rows), 0)
        cum = jnp.where(rid == _NB - 1, _BOUND, cum)
        left = jnp.concatenate(
            [jnp.full((1, rows), -_BOUND, jnp.float32), cum[:_NB - 1, :]], axis=0)
        return left, cum - left, cum

    cwl, wb, cwr = edges(uw, _MIN_W)
    chl, hb, _ = edges(uh, _MIN_H)

    d = _MIN_D + jax.nn.softplus(ud)
    rid33 = lax.broadcasted_iota(jnp.int32, (_NOUT - 2 * _NB, rows), 0)
    d = jnp.where((rid33 == 0) | (rid33 == _NB), 1.0, d)

    cnt = jnp.sum((xq >= cwr).astype(jnp.int32), axis=0, keepdims=True)
    idx = jnp.minimum(cnt, _NB - 1)

    onehot = (lax.broadcasted_iota(jnp.int32, (_NB, rows), 0)
              == idx).astype(jnp.float32)

    def pick(a):
        return jnp.sum(onehot * a, axis=0, keepdims=True)

    in_cw = pick(cwl)
    in_w = pick(wb)
    in_ch = pick(chl)
    in_h = pick(hb)
    d0 = pick(d[0:_NB, :])
    d1 = pick(d[1:_NB + 1, :])

    t = (xq - in_cw) / in_w
    tm = t * (1.0 - t)
    delta = in_h / in_w
    num = in_h * (delta * t * t + d0 * tm)
    den = delta + (d0 + d1 - 2.0 * delta) * tm
    outv = in_ch + num / den
    dnum = (delta * delta) * (d1 * t * t + 2.0 * delta * tm
                              + d0 * (1.0 - t) * (1.0 - t))
    lad = jnp.log(dnum) - 2.0 * jnp.log(den)

    z = jnp.where(inside, outv, xrow)
    jac = jnp.where(inside, lad, 0.0)
    prob = -_L2PI - 0.5 * z * z + jac

    p_hi = prob.astype(jnp.bfloat16)
    p_lo = (prob - p_hi.astype(jnp.float32)).astype(jnp.bfloat16)
    seg = s_ref[...]
    dn = (((1,), (0,)), ((), ()))
    psum = (lax.dot_general(p_hi, seg, dn, preferred_element_type=jnp.float32)
            + lax.dot_general(p_lo, seg, dn, preferred_element_type=jnp.float32))
    o_ref[...] = psum.reshape(1, 1, rows // _PART)


def kernel(x, c, W, b):
    nb, npart, _ = x.shape
    n = nb * npart
    grid = n // _ROWS
    bb = _ROWS // _PART

    be = _ROWS // npart
    b2 = b.reshape(-1, 1)
    seg = (jnp.arange(_ROWS, dtype=jnp.int32)[:, None] // _PART
           == jnp.arange(bb, dtype=jnp.int32)[None, :]).astype(jnp.bfloat16)

    out = pl.pallas_call(
        _flow_block,
        grid=(grid,),
        in_specs=[
            pl.BlockSpec((be, npart, c.shape[2]), lambda i: (i, 0, 0)),
            pl.BlockSpec(W.shape, lambda i: (0, 0)),
            pl.BlockSpec(b2.shape, lambda i: (0, 0)),
            pl.BlockSpec((1, 1, _ROWS), lambda i: (i, 0, 0)),
            pl.BlockSpec((_ROWS, bb), lambda i: (0, 0)),
        ],
        out_specs=pl.BlockSpec((1, 1, bb), lambda i: (i, 0, 0)),
        out_shape=jax.ShapeDtypeStruct((grid, 1, bb), jnp.float32),
        compiler_params=pltpu.CompilerParams(
            dimension_semantics=("parallel",),
        ),
    )(c, W, b2, x[..., -1].reshape(grid, 1, _ROWS), seg)
    return out.reshape(nb)


# speedup vs baseline: 1.0506x; 1.0506x over previous
import math

import jax
import jax.numpy as jnp
from jax import lax
from jax.experimental import pallas as pl
from jax.experimental.pallas import tpu as pltpu

_NB = 32
_NOUT = 3 * _NB + 1
_BOUND = 10.0
_MIN_W = 1e-05
_MIN_H = 1e-05
_MIN_D = 1e-05
_L2PI = 0.5 * math.log(2.0 * math.pi)
_PART = 16
_ROWS = 4096


def _flow_block(c_ref, w_ref, b_ref, x_ref, s_ref, o_ref):
    rows = c_ref.shape[1]
    theta = lax.dot_general(
        w_ref[...], c_ref[...],
        dimension_numbers=(((0,), (0,)), ((), ())),
        preferred_element_type=jnp.float32,
    ) + b_ref[...]

    uw = theta[0:_NB, :]
    uh = theta[_NB:2 * _NB, :]
    ud = theta[2 * _NB:_NOUT, :]

    xrow = x_ref[0]
    inside = (xrow >= -_BOUND) & (xrow <= _BOUND)
    xq = jnp.clip(xrow, -_BOUND, _BOUND)

    def edges(u, min_v):
        e = jnp.exp(u)
        scale = (1.0 - min_v * _NB) / jnp.sum(e, axis=0, keepdims=True)
        s = min_v + e * scale
        cum = s
        for k in (1, 2, 4, 8, 16):
            cum = cum + jnp.concatenate(
                [jnp.zeros((k, rows), jnp.float32), cum[:_NB - k, :]], axis=0)
        cum = 2.0 * _BOUND * cum - _BOUND
        rid = lax.broadcasted_iota(jnp.int32, (_NB, rows), 0)
        cum = jnp.where(rid == _NB - 1, _BOUND, cum)
        left = jnp.concatenate(
            [jnp.full((1, rows), -_BOUND, jnp.float32), cum[:_NB - 1, :]], axis=0)
        return left, cum - left, cum

    cwl, wb, cwr = edges(uw, _MIN_W)
    chl, hb, _ = edges(uh, _MIN_H)

    d = _MIN_D + jax.nn.softplus(ud)
    rid33 = lax.broadcasted_iota(jnp.int32, (_NOUT - 2 * _NB, rows), 0)
    d = jnp.where((rid33 == 0) | (rid33 == _NB), 1.0, d)

    cnt = jnp.sum((xq >= cwr).astype(jnp.int32), axis=0, keepdims=True)
    idx = jnp.minimum(cnt, _NB - 1)

    onehot = (lax.broadcasted_iota(jnp.int32, (_NB, rows), 0)
              == idx).astype(jnp.float32)

    def pick(a):
        return jnp.sum(onehot * a, axis=0, keepdims=True)

    in_cw = pick(cwl)
    in_w = pick(wb)
    in_ch = pick(chl)
    in_h = pick(hb)
    d0 = pick(d[0:_NB, :])
    d1 = pick(d[1:_NB + 1, :])

    t = (xq - in_cw) / in_w
    tm = t * (1.0 - t)
    delta = in_h / in_w
    num = in_h * (delta * t * t + d0 * tm)
    den = delta + (d0 + d1 - 2.0 * delta) * tm
    outv = in_ch + num / den
    dnum = (delta * delta) * (d1 * t * t + 2.0 * delta * tm
                              + d0 * (1.0 - t) * (1.0 - t))
    lad = jnp.log(dnum) - 2.0 * jnp.log(den)

    z = jnp.where(inside, outv, xrow)
    jac = jnp.where(inside, lad, 0.0)
    prob = -_L2PI - 0.5 * z * z + jac

    p_hi = prob.astype(jnp.bfloat16)
    p_lo = (prob - p_hi.astype(jnp.float32)).astype(jnp.bfloat16)
    seg = s_ref[...]
    dn = (((1,), (0,)), ((), ()))
    psum = (lax.dot_general(p_hi, seg, dn, preferred_element_type=jnp.float32)
            + lax.dot_general(p_lo, seg, dn, preferred_element_type=jnp.float32))
    o_ref[...] = psum.reshape(1, 1, rows // _PART)


def kernel(x, c, W, b):
    nb, npart, _ = x.shape
    n = nb * npart
    grid = n // _ROWS
    bb = _ROWS // _PART

    be = _ROWS // npart
    b2 = b.reshape(-1, 1)
    seg = (jnp.arange(_ROWS, dtype=jnp.int32)[:, None] // _PART
           == jnp.arange(bb, dtype=jnp.int32)[None, :]).astype(jnp.bfloat16)

    out = pl.pallas_call(
        _flow_block,
        grid=(grid,),
        in_specs=[
            pl.BlockSpec((c.shape[2], _ROWS), lambda i: (0, i)),
            pl.BlockSpec(W.shape, lambda i: (0, 0)),
            pl.BlockSpec(b2.shape, lambda i: (0, 0)),
            pl.BlockSpec((1, 1, _ROWS), lambda i: (i, 0, 0)),
            pl.BlockSpec((_ROWS, bb), lambda i: (0, 0)),
        ],
        out_specs=pl.BlockSpec((1, 1, bb), lambda i: (i, 0, 0)),
        out_shape=jax.ShapeDtypeStruct((grid, 1, bb), jnp.float32),
        compiler_params=pltpu.CompilerParams(
            dimension_semantics=("parallel",),
        ),
    )(c.reshape(n, -1).T, W, b2, x[..., -1].reshape(grid, 1, _ROWS), seg)
    return out.reshape(nb)


# speedup vs baseline: 1.4141x; 1.3460x over previous
import math

import jax
import jax.numpy as jnp
from jax import lax
from jax.experimental import pallas as pl
from jax.experimental.pallas import tpu as pltpu

_NB = 32
_NOUT = 3 * _NB + 1
_BOUND = 10.0
_MIN_W = 1e-05
_MIN_H = 1e-05
_MIN_D = 1e-05
_L2PI = 0.5 * math.log(2.0 * math.pi)
_PART = 16
_ROWS = 4096


def _flow_block(c_ref, w_ref, b_ref, x_ref, s_ref, o_ref):
    rows = c_ref.shape[0]
    theta = lax.dot_general(
        w_ref[...], c_ref[...],
        dimension_numbers=(((0,), (1,)), ((), ())),
        preferred_element_type=jnp.float32,
    ) + b_ref[...]

    uw = theta[0:_NB, :]
    uh = theta[_NB:2 * _NB, :]
    ud = theta[2 * _NB:_NOUT, :]

    xrow = x_ref[0]
    inside = (xrow >= -_BOUND) & (xrow <= _BOUND)
    xq = jnp.clip(xrow, -_BOUND, _BOUND)

    rid = lax.broadcasted_iota(jnp.int32, (_NB, rows), 0)

    def edges(u, min_v):
        e = jnp.exp(u)
        ecum = e
        for k in (1, 2, 4, 8, 16):
            ecum = ecum + jnp.concatenate(
                [jnp.zeros((k, rows), jnp.float32), ecum[:_NB - k, :]], axis=0)
        scale = (2.0 * _BOUND) * (1.0 - min_v * _NB) / ecum[_NB - 1:_NB, :]
        base = (2.0 * _BOUND * min_v) * (rid + 1).astype(jnp.float32) - _BOUND
        cum = base + ecum * scale
        cum = jnp.where(rid == _NB - 1, _BOUND, cum)
        left = jnp.concatenate(
            [jnp.full((1, rows), -_BOUND, jnp.float32), cum[:_NB - 1, :]], axis=0)
        return left, cum - left, cum

    cwl, wb, cwr = edges(uw, _MIN_W)
    chl, hb, _ = edges(uh, _MIN_H)

    one = jnp.ones((1, rows), jnp.float32)
    d = jnp.concatenate(
        [one, _MIN_D + jax.nn.softplus(ud[1:_NB, :]), one], axis=0)

    cnt = jnp.sum((xq >= cwr).astype(jnp.int32), axis=0, keepdims=True)
    idx = jnp.minimum(cnt, _NB - 1)

    onehot = (lax.broadcasted_iota(jnp.int32, (_NB, rows), 0)
              == idx).astype(jnp.float32)

    def pick(a):
        return jnp.sum(onehot * a, axis=0, keepdims=True)

    in_cw = pick(cwl)
    in_w = pick(wb)
    in_ch = pick(chl)
    in_h = pick(hb)
    d0 = pick(d[0:_NB, :])
    d1 = pick(d[1:_NB + 1, :])

    t = (xq - in_cw) / in_w
    tm = t * (1.0 - t)
    delta = in_h / in_w
    num = in_h * (delta * t * t + d0 * tm)
    den = delta + (d0 + d1 - 2.0 * delta) * tm
    outv = in_ch + num / den
    dnum = (delta * delta) * (d1 * t * t + 2.0 * delta * tm
                              + d0 * (1.0 - t) * (1.0 - t))
    lad = jnp.log(dnum) - 2.0 * jnp.log(den)

    z = jnp.where(inside, outv, xrow)
    jac = jnp.where(inside, lad, 0.0)
    prob = -_L2PI - 0.5 * z * z + jac

    p_hi = prob.astype(jnp.bfloat16)
    p_lo = (prob - p_hi.astype(jnp.float32)).astype(jnp.bfloat16)
    p2 = jnp.concatenate([p_hi, p_lo], axis=0)
    dn = (((1,), (0,)), ((), ()))
    ps2 = lax.dot_general(p2, s_ref[...], dn, preferred_element_type=jnp.float32)
    psum = ps2[0:1, :] + ps2[1:2, :]
    o_ref[...] = psum.reshape(1, 1, rows // _PART)


def kernel(x, c, W, b):
    nb, npart, _ = x.shape
    n = nb * npart
    grid = n // _ROWS
    bb = _ROWS // _PART

    be = _ROWS // npart
    b2 = b.reshape(-1, 1)
    seg = (jnp.arange(_ROWS, dtype=jnp.int32)[:, None] // _PART
           == jnp.arange(bb, dtype=jnp.int32)[None, :]).astype(jnp.bfloat16)

    out = pl.pallas_call(
        _flow_block,
        grid=(grid,),
        in_specs=[
            pl.BlockSpec((_ROWS, c.shape[2]), lambda i: (i, 0)),
            pl.BlockSpec(W.shape, lambda i: (0, 0)),
            pl.BlockSpec(b2.shape, lambda i: (0, 0)),
            pl.BlockSpec((1, 1, _ROWS), lambda i: (i, 0, 0)),
            pl.BlockSpec((_ROWS, bb), lambda i: (0, 0)),
        ],
        out_specs=pl.BlockSpec((1, 1, bb), lambda i: (i, 0, 0)),
        out_shape=jax.ShapeDtypeStruct((grid, 1, bb), jnp.float32),
        compiler_params=pltpu.CompilerParams(
            dimension_semantics=("parallel",),
        ),
    )(c.reshape(n, -1), W, b2, x[..., -1].reshape(grid, 1, _ROWS), seg)
    return out.reshape(nb)
